# baseline (speedup 1.0000x reference)
import functools

import jax
import jax.numpy as jnp
from jax import lax
from jax.experimental import pallas as pl
from jax.experimental.pallas import tpu as pltpu
from jax.experimental.pallas import tpu_sc as plsc

G = 1024
M = 64
N = 16384
QB = 128
BIG_I32 = 2**30


def _fps_body(start_ref, px_ref, py_ref, pz_ref, cx_ref, cy_ref, cz_ref):
    x = px_ref[:]
    y = py_ref[:]
    z = pz_ref[:]
    iota = (lax.broadcasted_iota(jnp.int32, (128, 128), 0) * 128
            + lax.broadcasted_iota(jnp.int32, (128, 128), 1))
    iota8 = (lax.broadcasted_iota(jnp.int32, (8, 128), 0) * 128
             + lax.broadcasted_iota(jnp.int32, (8, 128), 1))
    start = start_ref[0]

    m0 = iota == start
    zero = jnp.zeros((), jnp.float32)
    lx = jnp.sum(jnp.where(m0, x, 0.0))
    ly = jnp.sum(jnp.where(m0, y, 0.0))
    lz = jnp.sum(jnp.where(m0, z, 0.0))
    cx = jnp.where(iota8 == 0, lx, zero)
    cy = jnp.where(iota8 == 0, ly, zero)
    cz = jnp.where(iota8 == 0, lz, zero)
    dists = jnp.full((128, 128), jnp.inf, jnp.float32)

    def body(i, carry):
        dists, lx, ly, lz, cx, cy, cz = carry
        dx = x - lx
        dy = y - ly
        dz = z - lz
        d = (dx * dx + dy * dy) + dz * dz
        dists = jnp.minimum(dists, d)
        mx = jnp.max(dists)
        nxt = jnp.min(jnp.where(dists == mx, iota, BIG_I32))
        m = iota == nxt
        lx = jnp.sum(jnp.where(m, x, 0.0))
        ly = jnp.sum(jnp.where(m, y, 0.0))
        lz = jnp.sum(jnp.where(m, z, 0.0))
        sel = iota8 == i
        cx = jnp.where(sel, lx, cx)
        cy = jnp.where(sel, ly, cy)
        cz = jnp.where(sel, lz, cz)
        return dists, lx, ly, lz, cx, cy, cz

    carry = (dists, lx, ly, lz, cx, cy, cz)
    carry = lax.fori_loop(1, G, body, carry)
    _, _, _, _, cx, cy, cz = carry
    cx_ref[:] = cx
    cy_ref[:] = cy
    cz_ref[:] = cz


def _fps(px, py, pz, start):
    out = jax.ShapeDtypeStruct((8, 128), jnp.float32)
    return pl.pallas_call(
        _fps_body,
        out_shape=(out, out, out),
        in_specs=[
            pl.BlockSpec(memory_space=pltpu.MemorySpace.SMEM),
            pl.BlockSpec(memory_space=pltpu.MemorySpace.VMEM),
            pl.BlockSpec(memory_space=pltpu.MemorySpace.VMEM),
            pl.BlockSpec(memory_space=pltpu.MemorySpace.VMEM),
        ],
    )(start, px, py, pz)


_C = 128
_S = N // _C
_T = 6
_K = _C * _T
_KP = 128


def _topk_body(cx_ref, cy_ref, cz_ref, px_ref, py_ref, pz_ref, idx_ref,
               cxe_ref, cye_ref, cze_ref, d_ref, cvj_ref, cij_ref,
               lv_ref, li_ref):
    px = px_ref[:]
    py = py_ref[:]
    pz = pz_ref[:]
    qx = cx_ref[:]
    qy = cy_ref[:]
    qz = cz_ref[:]
    ones = jnp.ones((1, M), jnp.float32)
    cxe_ref[:] = qx * ones
    cye_ref[:] = qy * ones
    cze_ref[:] = qz * ones

    def bf(v):
        return v.astype(jnp.bfloat16).astype(jnp.float32)

    sump2 = (px * px + py * py) + pz * pz
    _DP = 16

    def dist_piece(p, _):
        s = pl.ds(p * _DP, _DP)
        qxp, qyp, qzp = cx_ref[s], cy_ref[s], cz_ref[s]
        mm = (bf(qxp) * bf(px) + bf(qyp) * bf(py)) + bf(qzp) * bf(pz)
        sumq2 = (qxp * qxp + qyp * qyp) + qzp * qzp
        d_ref[s] = ((sumq2 - 2.0 * mm) + sump2).reshape(_DP, _C, _S)
        return 0

    lax.fori_loop(0, QB // _DP, dist_piece, 0)

    lane64 = lax.broadcasted_iota(jnp.int32, (QB, M), 1)
    inf = jnp.float32(jnp.inf)

    _RP = 16

    def round_body(carry):
        lastv, lasti, m, out = carry
        act = m < M
        rem = M - m

        gp = (lax.broadcasted_iota(jnp.int32, (_RP, _C, _S), 1) * _S
              + lax.broadcasted_iota(jnp.int32, (_RP, _C, _S), 2))

        lv_ref[:] = lastv
        li_ref[:] = lasti

        def recompute_piece(p, _):
            dp = d_ref[pl.ds(p * _RP, _RP)]
            lv = lv_ref[pl.ds(p * _RP, _RP)][:, :, None]
            li = li_ref[pl.ds(p * _RP, _RP)][:, :, None]
            dm = jnp.where((dp > lv) | ((dp == lv) & (gp > li)), dp, inf)
            for t in range(_T):
                tv3 = jnp.min(dm, axis=2, keepdims=True)
                ti3 = jnp.min(jnp.where(dm == tv3, gp, BIG_I32), axis=2,
                              keepdims=True)
                cvj_ref[t, pl.ds(p * _RP, _RP), :] = jnp.min(dm, axis=2)
                cij_ref[t, pl.ds(p * _RP, _RP), :] = jnp.min(
                    jnp.where(dm == tv3, gp, BIG_I32), axis=2)
                if t < _T - 1:
                    dm = jnp.where((dm == tv3) & (gp == ti3), inf, dm)
            return 0

        lax.fori_loop(0, QB // _RP, recompute_piece, 0)

        _H = _C // 64

        def rank_half(t, h):
            av = cvj_ref[t][:, h * 64:(h + 1) * 64][:, :, None]
            ai = cij_ref[t][:, h * 64:(h + 1) * 64][:, :, None]

            def step(u, r):
                bv = cvj_ref[u][:, None, :]
                bi = cij_ref[u][:, None, :]
                less = (bv < av) | ((bv == av) & (bi < ai))
                return r + jnp.sum(less.astype(jnp.int32), axis=2,
                                   keepdims=True)

            return lax.fori_loop(0, _T, step,
                                 jnp.zeros((QB, 64, 1), jnp.int32)), av, ai

        rem3 = rem[:, :, None]
        stop = jnp.full((QB, 1), BIG_I32, jnp.int32)
        for h in range(_H):
            rd, _, _ = rank_half(_T - 1, h)
            stop = jnp.minimum(stop, jnp.min(
                jnp.where(rd < rem3, rd + 1, BIG_I32), axis=1))
        accepted = jnp.where(act, jnp.minimum(rem, stop), 0)
        acc3 = accepted[:, :, None]

        m3 = m[:, :, None]
        lane64_3 = lax.broadcasted_iota(jnp.int32, (QB, 1, M), 2)
        contrib = jnp.zeros((QB, M), jnp.int32)
        nlv = jnp.zeros((QB, 1), jnp.float32)
        nli = jnp.zeros((QB, 1), jnp.int32)

        def emit_level(t, carry):
            contrib, nlv, nli = carry
            for h in range(_H):
                r3, av, ai = rank_half(t, h)
                emit = r3 < acc3
                eqm = ((m3 + r3) == lane64_3) & emit
                contrib = contrib + jnp.sum(jnp.where(eqm, ai, 0), axis=1)
                lastsel = r3 == (acc3 - 1)
                nlv = nlv + jnp.sum(jnp.where(lastsel, av, 0.0), axis=1)
                nli = nli + jnp.sum(jnp.where(lastsel, ai, 0), axis=1)
            return contrib, nlv, nli

        contrib, nlv, nli = lax.fori_loop(0, _T, emit_level,
                                          (contrib, nlv, nli))
        out = out + contrib

        lastv = jnp.where(act, nlv, lastv)
        lasti = jnp.where(act, nli, lasti)
        m = m + accepted
        return lastv, lasti, m, out

    lastv = jnp.full((QB, 1), -jnp.inf, jnp.float32)
    lasti = jnp.full((QB, 1), -1, jnp.int32)
    m = jnp.zeros((QB, 1), jnp.int32)
    out = jnp.zeros((QB, M), jnp.int32)
    carry = lax.while_loop(lambda c: jnp.min(c[2]) < M, round_body,
                           (lastv, lasti, m, out))
    idx_ref[:] = carry[3]


def _topk(cxc, cyc, czc, px1, py1, pz1):
    return pl.pallas_call(
        _topk_body,
        grid=(G // QB,),
        out_shape=(
            jax.ShapeDtypeStruct((G, M), jnp.int32),
            jax.ShapeDtypeStruct((G, M), jnp.float32),
            jax.ShapeDtypeStruct((G, M), jnp.float32),
            jax.ShapeDtypeStruct((G, M), jnp.float32),
        ),
        in_specs=[
            pl.BlockSpec((QB, 1), lambda b: (b, 0)),
            pl.BlockSpec((QB, 1), lambda b: (b, 0)),
            pl.BlockSpec((QB, 1), lambda b: (b, 0)),
            pl.BlockSpec((1, N), lambda b: (0, 0)),
            pl.BlockSpec((1, N), lambda b: (0, 0)),
            pl.BlockSpec((1, N), lambda b: (0, 0)),
        ],
        out_specs=tuple(
            pl.BlockSpec((QB, M), lambda b: (b, 0)) for _ in range(4)),
        scratch_shapes=[pltpu.VMEM((QB, _C, _S), jnp.float32),
                        pltpu.VMEM((_T, QB, _C), jnp.float32),
                        pltpu.VMEM((_T, QB, _C), jnp.int32),
                        pltpu.VMEM((QB, 1), jnp.float32),
                        pltpu.VMEM((QB, 1), jnp.int32)],
    )(cxc, cyc, czc, px1, py1, pz1)


def _sc_gather(px, py, pz, idx, cxe, cye, cze):
    info = plsc.get_sparse_core_info()
    nc, ns, nl = info.num_cores, info.num_subcores, info.num_lanes
    nw = nc * ns
    b = G * M
    b_per_w = b // nw
    mesh = plsc.VectorSubcoreMesh(core_axis_name="c", subcore_axis_name="s")
    o = jax.ShapeDtypeStruct((b,), jnp.float32)

    @functools.partial(
        pl.kernel,
        mesh=mesh,
        compiler_params=pltpu.CompilerParams(needs_layout_passes=False),
        out_type=(o, o, o),
        scratch_types=[
            pltpu.VMEM((N,), jnp.float32),
            pltpu.VMEM((N,), jnp.float32),
            pltpu.VMEM((N,), jnp.float32),
            pltpu.VMEM((b_per_w,), jnp.int32),
            pltpu.VMEM((b_per_w,), jnp.float32),
            pltpu.VMEM((b_per_w,), jnp.float32),
            pltpu.VMEM((b_per_w,), jnp.float32),
            pltpu.VMEM((b_per_w,), jnp.float32),
            pltpu.VMEM((b_per_w,), jnp.float32),
            pltpu.VMEM((b_per_w,), jnp.float32),
        ],
    )
    def k(px_hbm, py_hbm, pz_hbm, idx_hbm, cxe_hbm, cye_hbm, cze_hbm,
          ox_hbm, oy_hbm, oz_hbm,
          x_v, y_v, z_v, idx_v, cx_v, cy_v, cz_v, ox_v, oy_v, oz_v):
        wid = lax.axis_index("s") * nc + lax.axis_index("c")
        base = wid * b_per_w
        pltpu.sync_copy(px_hbm, x_v)
        pltpu.sync_copy(py_hbm, y_v)
        pltpu.sync_copy(pz_hbm, z_v)
        pltpu.sync_copy(idx_hbm.at[pl.ds(base, b_per_w)], idx_v)
        pltpu.sync_copy(cxe_hbm.at[pl.ds(base, b_per_w)], cx_v)
        pltpu.sync_copy(cye_hbm.at[pl.ds(base, b_per_w)], cy_v)
        pltpu.sync_copy(cze_hbm.at[pl.ds(base, b_per_w)], cz_v)

        def body(i, _):
            s = pl.ds(i * nl, nl)
            iv = idx_v[s]
            ox_v[s] = plsc.load_gather(x_v, [iv]) - cx_v[s]
            oy_v[s] = plsc.load_gather(y_v, [iv]) - cy_v[s]
            oz_v[s] = plsc.load_gather(z_v, [iv]) - cz_v[s]
            return 0

        lax.fori_loop(0, b_per_w // nl, body, 0)
        pltpu.sync_copy(ox_v, ox_hbm.at[pl.ds(base, b_per_w)])
        pltpu.sync_copy(oy_v, oy_hbm.at[pl.ds(base, b_per_w)])
        pltpu.sync_copy(oz_v, oz_hbm.at[pl.ds(base, b_per_w)])

    return k(px, py, pz, idx, cxe, cye, cze)


def kernel(pc, key):
    start = jax.random.randint(key, (), 0, N).astype(jnp.int32)
    px = pc[:, 0].reshape(128, 128)
    py = pc[:, 1].reshape(128, 128)
    pz = pc[:, 2].reshape(128, 128)
    cx, cy, cz = _fps(px, py, pz, start.reshape(1))

    idx, cxe, cye, cze = _topk(
        cx.reshape(G, 1), cy.reshape(G, 1), cz.reshape(G, 1),
        pc[:, 0].reshape(1, N), pc[:, 1].reshape(1, N), pc[:, 2].reshape(1, N),
    )

    center = jnp.stack([cx.reshape(G), cy.reshape(G), cz.reshape(G)], axis=-1)
    ox, oy, oz = _sc_gather(
        pc[:, 0], pc[:, 1], pc[:, 2], idx.reshape(G * M),
        cxe.reshape(G * M), cye.reshape(G * M), cze.reshape(G * M))
    neighborhood = jnp.stack([ox, oy, oz], axis=-1).reshape(G, M, 3)
    return (neighborhood, center)


# speedup vs baseline: 1.9379x; 1.9379x over previous
import functools

import jax
import jax.numpy as jnp
from jax import lax
from jax.experimental import pallas as pl
from jax.experimental.pallas import tpu as pltpu
from jax.experimental.pallas import tpu_sc as plsc

G = 1024
M = 64
N = 16384
QB = 128
BIG_I32 = 2**30


def _fps_body(start_ref, px_ref, py_ref, pz_ref, cx_ref, cy_ref, cz_ref):
    x = px_ref[:]
    y = py_ref[:]
    z = pz_ref[:]
    iota = (lax.broadcasted_iota(jnp.int32, (128, 128), 0) * 128
            + lax.broadcasted_iota(jnp.int32, (128, 128), 1))
    iota8 = (lax.broadcasted_iota(jnp.int32, (8, 128), 0) * 128
             + lax.broadcasted_iota(jnp.int32, (8, 128), 1))
    start = start_ref[0]

    m0 = iota == start
    zero = jnp.zeros((), jnp.float32)
    lx = jnp.sum(jnp.where(m0, x, 0.0))
    ly = jnp.sum(jnp.where(m0, y, 0.0))
    lz = jnp.sum(jnp.where(m0, z, 0.0))
    cx = jnp.where(iota8 == 0, lx, zero)
    cy = jnp.where(iota8 == 0, ly, zero)
    cz = jnp.where(iota8 == 0, lz, zero)
    dists = jnp.full((128, 128), jnp.inf, jnp.float32)

    def body(i, carry):
        dists, lx, ly, lz, cx, cy, cz = carry
        dx = x - lx
        dy = y - ly
        dz = z - lz
        d = (dx * dx + dy * dy) + dz * dz
        dists = jnp.minimum(dists, d)
        mx = jnp.max(dists)
        nxt = jnp.min(jnp.where(dists == mx, iota, BIG_I32))
        m = iota == nxt
        lx = jnp.sum(jnp.where(m, x, 0.0))
        ly = jnp.sum(jnp.where(m, y, 0.0))
        lz = jnp.sum(jnp.where(m, z, 0.0))
        sel = iota8 == i
        cx = jnp.where(sel, lx, cx)
        cy = jnp.where(sel, ly, cy)
        cz = jnp.where(sel, lz, cz)
        return dists, lx, ly, lz, cx, cy, cz

    carry = (dists, lx, ly, lz, cx, cy, cz)
    carry = lax.fori_loop(1, G, body, carry)
    _, _, _, _, cx, cy, cz = carry
    cx_ref[:] = cx
    cy_ref[:] = cy
    cz_ref[:] = cz


def _fps(px, py, pz, start):
    out = jax.ShapeDtypeStruct((8, 128), jnp.float32)
    return pl.pallas_call(
        _fps_body,
        out_shape=(out, out, out),
        in_specs=[
            pl.BlockSpec(memory_space=pltpu.MemorySpace.SMEM),
            pl.BlockSpec(memory_space=pltpu.MemorySpace.VMEM),
            pl.BlockSpec(memory_space=pltpu.MemorySpace.VMEM),
            pl.BlockSpec(memory_space=pltpu.MemorySpace.VMEM),
        ],
    )(start, px, py, pz)


_C = 512
_S = N // _C
_T = 4


def _topk_body(cx_ref, cy_ref, cz_ref, px_ref, py_ref, pz_ref, idx_ref,
               cxe_ref, cye_ref, cze_ref, d_ref):
    px = px_ref[:]
    py = py_ref[:]
    pz = pz_ref[:]
    qx = cx_ref[:]
    qy = cy_ref[:]
    qz = cz_ref[:]
    ones = jnp.ones((1, M), jnp.float32)
    cxe_ref[:] = qx * ones
    cye_ref[:] = qy * ones
    cze_ref[:] = qz * ones

    def bf(v):
        return v.astype(jnp.bfloat16).astype(jnp.float32)

    mm = (bf(qx) * bf(px) + bf(qy) * bf(py)) + bf(qz) * bf(pz)
    sumq2 = (qx * qx + qy * qy) + qz * qz
    sump2 = (px * px + py * py) + pz * pz
    d_ref[:] = ((sumq2 - 2.0 * mm) + sump2).reshape(QB, _C, _S)

    lane64 = lax.broadcasted_iota(jnp.int32, (QB, M), 1)
    inf = jnp.float32(jnp.inf)

    def extract(carry):
        taken, tv, ti, lastv, lasti, m, out, _ = carry
        hv = tv[_T - 1]
        hi = ti[_T - 1]
        for t in range(_T - 2, -1, -1):
            hv = jnp.where(taken == t, tv[t], hv)
            hi = jnp.where(taken == t, ti[t], hi)
        hv = jnp.where(taken >= _T, inf, hv)
        mnv = jnp.min(hv, axis=1, keepdims=True)
        wini = jnp.min(jnp.where(hv == mnv, hi, BIG_I32), axis=1,
                       keepdims=True)
        unsafe = ((taken >= _T)
                  & ((tv[_T - 1] < mnv)
                     | ((tv[_T - 1] == mnv) & (ti[_T - 1] < wini))))
        blocked = jnp.max(unsafe.astype(jnp.int32), axis=1,
                          keepdims=True) > 0
        act = (mnv < inf) & jnp.logical_not(blocked) & (m < M)
        out = jnp.where((lane64 == m) & act, wini, out)
        wc = (hv == mnv) & (hi == wini) & act
        taken = taken + jnp.where(wc, 1, 0)
        lastv = jnp.where(act, mnv, lastv)
        lasti = jnp.where(act, wini, lasti)
        m = m + jnp.where(act, 1, 0)
        anyact = jnp.max(act.astype(jnp.int32)) > 0
        return taken, tv, ti, lastv, lasti, m, out, anyact

    _CP = 64

    def piece_topT(p, lastv, lasti):
        dp = d_ref[:, p * _CP:(p + 1) * _CP, :]
        gp = (lax.broadcasted_iota(jnp.int32, (QB, _CP, _S), 1)
              * _S + lax.broadcasted_iota(jnp.int32, (QB, _CP, _S), 2)
              + p * _CP * _S)
        lv = lastv[:, :, None]
        li = lasti[:, :, None]
        dm = jnp.where((dp > lv) | ((dp == lv) & (gp > li)), dp, inf)
        vs, isx = [], []
        for t in range(_T):
            tv = jnp.min(dm, axis=2)
            ti = jnp.min(jnp.where(dm == tv[:, :, None], gp, BIG_I32), axis=2)
            vs.append(tv)
            isx.append(ti)
            if t < _T - 1:
                dm = jnp.where((dm == tv[:, :, None]) & (gp == ti[:, :, None]),
                               inf, dm)
        return vs, isx

    def round_body(carry):
        lastv, lasti, m, out = carry
        parts = [piece_topT(p, lastv, lasti) for p in range(_C // _CP)]
        tv = [jnp.concatenate([x[0][t] for x in parts], axis=1)
              for t in range(_T)]
        ti = [jnp.concatenate([x[1][t] for x in parts], axis=1)
              for t in range(_T)]
        taken = jnp.zeros((QB, _C), jnp.int32)
        st = (taken, tuple(tv), tuple(ti), lastv, lasti, m, out,
              jnp.bool_(True))
        st = lax.while_loop(lambda c: c[7], extract, st)
        return st[3], st[4], st[5], st[6]

    lastv = jnp.full((QB, 1), -jnp.inf, jnp.float32)
    lasti = jnp.full((QB, 1), -1, jnp.int32)
    m = jnp.zeros((QB, 1), jnp.int32)
    out = jnp.zeros((QB, M), jnp.int32)
    carry = lax.while_loop(lambda c: jnp.min(c[2]) < M, round_body,
                           (lastv, lasti, m, out))
    idx_ref[:] = carry[3]


def _topk(cxc, cyc, czc, px1, py1, pz1):
    return pl.pallas_call(
        _topk_body,
        grid=(G // QB,),
        out_shape=(
            jax.ShapeDtypeStruct((G, M), jnp.int32),
            jax.ShapeDtypeStruct((G, M), jnp.float32),
            jax.ShapeDtypeStruct((G, M), jnp.float32),
            jax.ShapeDtypeStruct((G, M), jnp.float32),
        ),
        in_specs=[
            pl.BlockSpec((QB, 1), lambda b: (b, 0)),
            pl.BlockSpec((QB, 1), lambda b: (b, 0)),
            pl.BlockSpec((QB, 1), lambda b: (b, 0)),
            pl.BlockSpec((1, N), lambda b: (0, 0)),
            pl.BlockSpec((1, N), lambda b: (0, 0)),
            pl.BlockSpec((1, N), lambda b: (0, 0)),
        ],
        out_specs=tuple(
            pl.BlockSpec((QB, M), lambda b: (b, 0)) for _ in range(4)),
        scratch_shapes=[pltpu.VMEM((QB, _C, _S), jnp.float32)],
    )(cxc, cyc, czc, px1, py1, pz1)


def _sc_gather(px, py, pz, idx, cxe, cye, cze):
    info = plsc.get_sparse_core_info()
    nc, ns, nl = info.num_cores, info.num_subcores, info.num_lanes
    nw = nc * ns
    b = G * M
    b_per_w = b // nw
    mesh = plsc.VectorSubcoreMesh(core_axis_name="c", subcore_axis_name="s")
    o = jax.ShapeDtypeStruct((b,), jnp.float32)

    @functools.partial(
        pl.kernel,
        mesh=mesh,
        compiler_params=pltpu.CompilerParams(needs_layout_passes=False),
        out_type=(o, o, o),
        scratch_types=[
            pltpu.VMEM((N,), jnp.float32),
            pltpu.VMEM((N,), jnp.float32),
            pltpu.VMEM((N,), jnp.float32),
            pltpu.VMEM((b_per_w,), jnp.int32),
            pltpu.VMEM((b_per_w,), jnp.float32),
            pltpu.VMEM((b_per_w,), jnp.float32),
            pltpu.VMEM((b_per_w,), jnp.float32),
            pltpu.VMEM((b_per_w,), jnp.float32),
            pltpu.VMEM((b_per_w,), jnp.float32),
            pltpu.VMEM((b_per_w,), jnp.float32),
        ],
    )
    def k(px_hbm, py_hbm, pz_hbm, idx_hbm, cxe_hbm, cye_hbm, cze_hbm,
          ox_hbm, oy_hbm, oz_hbm,
          x_v, y_v, z_v, idx_v, cx_v, cy_v, cz_v, ox_v, oy_v, oz_v):
        wid = lax.axis_index("s") * nc + lax.axis_index("c")
        base = wid * b_per_w
        pltpu.sync_copy(px_hbm, x_v)
        pltpu.sync_copy(py_hbm, y_v)
        pltpu.sync_copy(pz_hbm, z_v)
        pltpu.sync_copy(idx_hbm.at[pl.ds(base, b_per_w)], idx_v)
        pltpu.sync_copy(cxe_hbm.at[pl.ds(base, b_per_w)], cx_v)
        pltpu.sync_copy(cye_hbm.at[pl.ds(base, b_per_w)], cy_v)
        pltpu.sync_copy(cze_hbm.at[pl.ds(base, b_per_w)], cz_v)

        def body(i, _):
            s = pl.ds(i * nl, nl)
            iv = idx_v[s]
            ox_v[s] = plsc.load_gather(x_v, [iv]) - cx_v[s]
            oy_v[s] = plsc.load_gather(y_v, [iv]) - cy_v[s]
            oz_v[s] = plsc.load_gather(z_v, [iv]) - cz_v[s]
            return 0

        lax.fori_loop(0, b_per_w // nl, body, 0)
        pltpu.sync_copy(ox_v, ox_hbm.at[pl.ds(base, b_per_w)])
        pltpu.sync_copy(oy_v, oy_hbm.at[pl.ds(base, b_per_w)])
        pltpu.sync_copy(oz_v, oz_hbm.at[pl.ds(base, b_per_w)])

    return k(px, py, pz, idx, cxe, cye, cze)


def kernel(pc, key):
    start = jax.random.randint(key, (), 0, N).astype(jnp.int32)
    px = pc[:, 0].reshape(128, 128)
    py = pc[:, 1].reshape(128, 128)
    pz = pc[:, 2].reshape(128, 128)
    cx, cy, cz = _fps(px, py, pz, start.reshape(1))

    idx, cxe, cye, cze = _topk(
        cx.reshape(G, 1), cy.reshape(G, 1), cz.reshape(G, 1),
        pc[:, 0].reshape(1, N), pc[:, 1].reshape(1, N), pc[:, 2].reshape(1, N),
    )

    center = jnp.stack([cx.reshape(G), cy.reshape(G), cz.reshape(G)], axis=-1)
    ox, oy, oz = _sc_gather(
        pc[:, 0], pc[:, 1], pc[:, 2], idx.reshape(G * M),
        cxe.reshape(G * M), cye.reshape(G * M), cze.reshape(G * M))
    neighborhood = jnp.stack([ox, oy, oz], axis=-1).reshape(G, M, 3)
    return (neighborhood, center)
